# baseline (speedup 1.0000x reference)
import functools

import jax
import jax.numpy as jnp
from jax.experimental import pallas as pl
from jax.experimental.pallas import tpu as pltpu

_DELTA = 0.5
_LOSS_WEIGHT = 0.5
_TOPK = 64
_SOFT = 6.0
_NEG_INF = float("-inf")


def _topk_body(nc, pred_t_ref, tgt_rep_ref, fp_ref):
    n, bc = pred_t_ref.shape
    keep = tgt_rep_ref[...] == nc
    a0 = jnp.where(keep, pred_t_ref[...], _NEG_INF)
    row_iota = jax.lax.broadcasted_iota(jnp.int32, (n, bc), 0)

    def step(k, a):
        m = jnp.max(a, axis=0, keepdims=True)
        hit = a == m
        first = jnp.min(jnp.where(hit, row_iota, n), axis=0, keepdims=True)
        fp_ref[pl.ds(k, 1), :] = m
        return jnp.where(row_iota == first, _NEG_INF, a)

    jax.lax.fori_loop(0, _TOPK, step, a0)


def _pair_body(tp_ref, tgt_ref, fp_ref, s_ref, np_ref):
    step = pl.program_id(0)
    fp = fp_ref[...]
    w = jnp.where(jnp.isinf(fp), 0.0, 1.0)
    tp = tp_ref[...]
    x = (fp[None] - tp[:, :, None] + _DELTA) * w[None]
    f = jnp.log(1.0 + jnp.exp(_SOFT * x)) * (1.0 / _SOFT)
    rows = jnp.sum(f, axis=(1, 2))
    m = (tgt_ref[...][:, 0] != tp_ref.shape[1]).astype(jnp.float32)
    s_step = jnp.sum(rows * m)
    np_step = jnp.sum(m)

    @pl.when(step == 0)
    def _():
        s_ref[...] = jnp.zeros((1, 1), jnp.float32)
        np_ref[...] = jnp.zeros((1, 1), jnp.float32)

    s_ref[...] += s_step.reshape(1, 1)
    np_ref[...] += np_step.reshape(1, 1)


@jax.jit
def kernel(pred, target):
    bsz, n, c = pred.shape
    bc = bsz * c
    bk = bsz * _TOPK

    pred_t = jnp.transpose(pred, (1, 0, 2)).reshape(n, bc)
    tgt_rep = jnp.broadcast_to(
        jnp.transpose(target, (1, 0))[:, :, None], (n, bsz, c)
    ).reshape(n, bc)

    fp_kbc = pl.pallas_call(
        functools.partial(_topk_body, c),
        out_shape=jax.ShapeDtypeStruct((_TOPK, bc), jnp.float32),
    )(pred_t, tgt_rep)

    fp_cbk = jnp.transpose(fp_kbc.reshape(_TOPK, bsz, c), (2, 1, 0)).reshape(c, bk)

    rows = bsz * n
    blk = 512
    grid = rows // blk
    s, npos = pl.pallas_call(
        _pair_body,
        grid=(grid,),
        in_specs=[
            pl.BlockSpec((blk, c), lambda i: (i, 0)),
            pl.BlockSpec((blk, 1), lambda i: (i, 0)),
            pl.BlockSpec((c, bk), lambda i: (0, 0)),
        ],
        out_specs=[
            pl.BlockSpec((1, 1), lambda i: (0, 0)),
            pl.BlockSpec((1, 1), lambda i: (0, 0)),
        ],
        out_shape=[
            jax.ShapeDtypeStruct((1, 1), jnp.float32),
            jax.ShapeDtypeStruct((1, 1), jnp.float32),
        ],
    )(pred.reshape(rows, c), target.reshape(rows, 1), fp_cbk)

    denom = npos[0, 0] * float(bk * c)
    return _LOSS_WEIGHT * s[0, 0] / denom


# speedup vs baseline: 1.0544x; 1.0544x over previous
import functools

import jax
import jax.numpy as jnp
from jax.experimental import pallas as pl
from jax.experimental.pallas import tpu as pltpu

_DELTA = 0.5
_LOSS_WEIGHT = 0.5
_TOPK = 64
_SOFT = 6.0
_NEG_INF = float("-inf")
_K6 = _SOFT * 1.4426950408889634
_LN2 = 0.6931471805599453


def _topk_body(nc, pred_t_ref, tgt_rep_ref, fp_ref):
    n, bc = pred_t_ref.shape
    keep = tgt_rep_ref[...] == nc
    a0 = jnp.where(keep, pred_t_ref[...], _NEG_INF)
    row_iota = jax.lax.broadcasted_iota(jnp.int32, (n, bc), 0)

    def step(k, a):
        m = jnp.max(a, axis=0, keepdims=True)
        hit = a == m
        first = jnp.min(jnp.where(hit, row_iota, n), axis=0, keepdims=True)
        fp_ref[pl.ds(k, 1), :] = jnp.where(
            jnp.isinf(m), jnp.nan, _K6 * (m + _DELTA)
        )
        return jnp.where(row_iota == first, _NEG_INF, a)

    jax.lax.fori_loop(0, _TOPK, step, a0)


def _pair_body(tp_ref, tgt_ref, a2_ref, s_ref, np_ref):
    step = pl.program_id(0)
    a2 = a2_ref[...]
    tp6 = tp_ref[...] * _K6
    x2 = a2[None] - tp6[:, :, None]
    f2 = jnp.log2(1.0 + jnp.exp2(x2))
    rows = jnp.sum(f2, axis=(1, 2))
    m = (tgt_ref[...][:, 0] != tp_ref.shape[1]).astype(jnp.float32)
    s_step = jnp.sum(rows * m)
    np_step = jnp.sum(m)

    @pl.when(step == 0)
    def _():
        s_ref[...] = jnp.zeros((1, 1), jnp.float32)
        np_ref[...] = jnp.zeros((1, 1), jnp.float32)

    s_ref[...] += s_step.reshape(1, 1)
    np_ref[...] += np_step.reshape(1, 1)


@jax.jit
def kernel(pred, target):
    bsz, n, c = pred.shape
    bc = bsz * c
    bk = bsz * _TOPK

    pred_t = jnp.transpose(pred, (1, 0, 2)).reshape(n, bc)
    tgt_rep = jnp.broadcast_to(
        jnp.transpose(target, (1, 0))[:, :, None], (n, bsz, c)
    ).reshape(n, bc)

    fp_kbc = pl.pallas_call(
        functools.partial(_topk_body, c),
        out_shape=jax.ShapeDtypeStruct((_TOPK, bc), jnp.float32),
    )(pred_t, tgt_rep)

    fp_cbk = jnp.transpose(fp_kbc.reshape(_TOPK, bsz, c), (2, 1, 0)).reshape(c, bk)

    rows = bsz * n
    blk = 512
    grid = rows // blk
    s, npos = pl.pallas_call(
        _pair_body,
        grid=(grid,),
        in_specs=[
            pl.BlockSpec((blk, c), lambda i: (i, 0)),
            pl.BlockSpec((blk, 1), lambda i: (i, 0)),
            pl.BlockSpec((c, bk), lambda i: (0, 0)),
        ],
        out_specs=[
            pl.BlockSpec((1, 1), lambda i: (0, 0)),
            pl.BlockSpec((1, 1), lambda i: (0, 0)),
        ],
        out_shape=[
            jax.ShapeDtypeStruct((1, 1), jnp.float32),
            jax.ShapeDtypeStruct((1, 1), jnp.float32),
        ],
    )(pred.reshape(rows, c), target.reshape(rows, 1), fp_cbk)

    denom = npos[0, 0] * float(bk * c)
    return (_LOSS_WEIGHT * _LN2 / _SOFT) * s[0, 0] / denom
